# baseline (speedup 1.0000x reference)
import jax
import jax.numpy as jnp
from jax import lax
from jax.experimental import pallas as pl
from jax.experimental.pallas import tpu as pltpu
from jax.experimental.pallas import tpu_sc as plsc

N = 10000
NPAD = 10240
E = 320000
NC = 2
NS = 16
NW = NC * NS
RPT = NPAD // NS
EPW = E // NW
B = 125
NCHUNK = EPW // B

_f32 = jnp.float32


def _make_seg_sum(D, with_deg):
  mesh = plsc.VectorSubcoreMesh(core_axis_name="c", subcore_axis_name="s")
  out_type = [jax.ShapeDtypeStruct((NC, NPAD, D), _f32)]
  scratch = [
      pltpu.VMEM((NCHUNK, B), jnp.int32),
      pltpu.VMEM((NCHUNK, B), jnp.int32),
      pltpu.VMEM((B, D), _f32),
      pltpu.VMEM((B, D), _f32),
      pltpu.VMEM_SHARED((NPAD, D), _f32),
      pltpu.SemaphoreType.DMA,
      pltpu.SemaphoreType.DMA,
  ]
  if with_deg:
    out_type.append(jax.ShapeDtypeStruct((NC, NPAD), _f32))
    scratch += [
        pltpu.VMEM((128,), _f32),
        pltpu.VMEM_SHARED((NPAD,), _f32),
        pltpu.SemaphoreType.DMA,
    ]

  def body(table_hbm, ei_hbm, zeros_hbm, *rest):
    if with_deg:
      (zd_hbm, acc_out, deg_out, src_v, dst_v, rows0, rows1, acc_sh,
       sem0, sem1, ones_v, deg_sh, semd) = rest
    else:
      (acc_out, src_v, dst_v, rows0, rows1, acc_sh,
       sem0, sem1) = rest
    c = lax.axis_index("c")
    s = lax.axis_index("s")
    wid = c * NS + s
    row0 = pl.multiple_of(s * RPT, 8)

    ldsrc = pltpu.async_copy(ei_hbm.at[0, wid], src_v, sem1)
    pltpu.sync_copy(zeros_hbm, acc_sh.at[pl.ds(row0, RPT)])
    pltpu.sync_copy(ei_hbm.at[1, wid], dst_v)
    if with_deg:
      ones16 = jnp.ones((16,), _f32)
      for k in range(128 // 16):
        ones_v[pl.ds(k * 16, 16)] = ones16
      pltpu.sync_copy(zd_hbm, deg_sh.at[pl.ds(row0, RPT)])
    ldsrc.wait()
    plsc.subcore_barrier()

    def g_start(j, buf, sem):
      pltpu.async_copy(table_hbm.at[src_v.at[j]], buf, sem)

    def g_wait(j, buf, sem):
      pltpu.make_async_copy(table_hbm.at[src_v.at[j]], buf, sem).wait()

    def consume(j, buf):
      pltpu.sync_copy(buf, acc_sh.at[dst_v.at[j]], add=True)
      if with_deg:
        pltpu.async_copy(ones_v.at[pl.ds(0, B)], deg_sh.at[dst_v.at[j]],
                        semd, add=True)

    g_start(0, rows0, sem0)
    g_start(1, rows1, sem1)

    def step(i, carry):
      j0 = 2 * i
      g_wait(j0, rows0, sem0)
      consume(j0, rows0)
      g_start(j0 + 2, rows0, sem0)
      g_wait(j0 + 1, rows1, sem1)
      consume(j0 + 1, rows1)
      g_start(j0 + 3, rows1, sem1)
      return carry

    lax.fori_loop(0, NCHUNK // 2 - 1, step, 0)
    g_wait(NCHUNK - 2, rows0, sem0)
    consume(NCHUNK - 2, rows0)
    g_wait(NCHUNK - 1, rows1, sem1)
    consume(NCHUNK - 1, rows1)
    if with_deg:
      def drain(j, carry):
        pltpu.make_async_copy(ones_v.at[pl.ds(0, B)],
                              deg_sh.at[dst_v.at[j]], semd).wait()
        return carry
      lax.fori_loop(0, NCHUNK, drain, 0)
    plsc.subcore_barrier()

    pltpu.sync_copy(acc_sh.at[pl.ds(row0, RPT)],
                    acc_out.at[c, pl.ds(row0, RPT)])
    if with_deg:
      pltpu.sync_copy(deg_sh.at[pl.ds(row0, RPT)],
                      deg_out.at[c, pl.ds(row0, RPT)])

  return pl.kernel(body, out_type=out_type, mesh=mesh, scratch_types=scratch,
                   compiler_params=pltpu.CompilerParams(
                       use_tc_tiling_on_sc=False))


_seg_sum_l1 = _make_seg_sum(32, with_deg=True)
_seg_sum_l2 = _make_seg_sum(16, with_deg=False)

BN = 1280
_ONES2 = (((0,), (0,)), ((), ()))


def _rdeg_col(deg2):
  ones2 = jnp.ones((2, 1), _f32)
  deg = lax.dot_general(deg2, ones2, _ONES2, preferred_element_type=_f32)
  return 1.0 / jnp.maximum(deg, 1.0)


def _stage1_body(x_ref, wl_ref, wr_ref, t_o, r_o):
  xv = x_ref[...]
  t_o[...] = jnp.dot(xv, wl_ref[...], preferred_element_type=_f32)
  r_o[...] = jnp.dot(xv, wr_ref[...], preferred_element_type=_f32)


def _mid_body(acc_ref, deg_ref, p1r_ref, b1_ref, w2l_ref, w2r_ref,
              t_o, p2r_o):
  acc = acc_ref[...]
  s1 = acc[0] + acc[1]
  rdeg = _rdeg_col(deg_ref[...])
  h = s1 * rdeg + b1_ref[...] + p1r_ref[...]
  h = jnp.maximum(h, 0.0)
  t_o[...] = jnp.dot(h, w2l_ref[...], preferred_element_type=_f32)
  p2r_o[...] = jnp.dot(h, w2r_ref[...], preferred_element_type=_f32)


def _dec_body(acc_ref, deg_ref, p2r_ref, b2_ref, wd_ref, bd_ref,
              xr_o, z_o):
  acc = acc_ref[...]
  rdeg = _rdeg_col(deg_ref[...])
  z = (acc[0] + acc[1]) * rdeg + b2_ref[...] + p2r_ref[...]
  z_o[...] = z
  xr_o[...] = jnp.dot(z, wd_ref[...], preferred_element_type=_f32) + bd_ref[...]


def _row_blocks(width, bn=BN):
  return pl.BlockSpec((bn, width), lambda i: (i, 0))


def _acc_blocks(width, bn=BN):
  return pl.BlockSpec((NC, bn, width), lambda i: (0, i, 0))


def _deg_blocks(bn=BN):
  return pl.BlockSpec((NC, bn), lambda i: (0, i))


def _full(shape):
  return pl.BlockSpec(shape, lambda i: tuple(0 for _ in shape))


def kernel(x, edge_index, W1_l, b1_l, W1_r, W2_l, b2_l, W2_r, W_dec, b_dec):
  ei = lax.convert_element_type(edge_index, jnp.int32).reshape(
      2, NW, NCHUNK, B)
  z1 = jnp.zeros((RPT, 32), _f32)
  z2 = jnp.zeros((RPT, 16), _f32)

  table1, p1r = pl.pallas_call(
      _stage1_body,
      grid=(NPAD // BN,),
      in_specs=[_row_blocks(128), _full((128, 32)), _full((128, 32))],
      out_specs=[_row_blocks(32), _row_blocks(32)],
      out_shape=[jax.ShapeDtypeStruct((NPAD, 32), _f32),
                 jax.ShapeDtypeStruct((NPAD, 32), _f32)],
  )(x, W1_l.T, W1_r.T)

  acc1, degp = _seg_sum_l1(table1, ei, z1, jnp.zeros((RPT,), _f32))

  table2, p2r = pl.pallas_call(
      _mid_body,
      grid=(NPAD // BN,),
      in_specs=[_acc_blocks(32), _deg_blocks(), _row_blocks(32),
                _full((1, 32)), _full((32, 16)), _full((32, 16))],
      out_specs=[_row_blocks(16), _row_blocks(16)],
      out_shape=[jax.ShapeDtypeStruct((NPAD, 16), _f32),
                 jax.ShapeDtypeStruct((NPAD, 16), _f32)],
  )(acc1, degp, p1r, b1_l[None, :], W2_l.T, W2_r.T)

  (acc2,) = _seg_sum_l2(table2, ei, z2)

  x_recon, z = pl.pallas_call(
      _dec_body,
      grid=(NPAD // BN,),
      in_specs=[_acc_blocks(16), _deg_blocks(), _row_blocks(16),
                _full((1, 16)), _full((16, 128)), _full((1, 128))],
      out_specs=[_row_blocks(128), _row_blocks(16)],
      out_shape=[jax.ShapeDtypeStruct((N, 128), _f32),
                 jax.ShapeDtypeStruct((N, 16), _f32)],
  )(acc2, degp, p2r, b2_l[None, :], W_dec.T, b_dec[None, :])

  return (x_recon, z)


# speedup vs baseline: 1.2107x; 1.2107x over previous
import jax
import jax.numpy as jnp
from jax import lax
from jax.experimental import pallas as pl
from jax.experimental.pallas import tpu as pltpu
from jax.experimental.pallas import tpu_sc as plsc

N = 10000
NPAD = 10240
E = 320000
NC = 2
NS = 16
NW = NC * NS
RPT = NPAD // NS
EPW = E // NW
B = 125
NCHUNK = EPW // B

_f32 = jnp.float32


def _make_seg_sum(D, with_deg):
  mesh = plsc.VectorSubcoreMesh(core_axis_name="c", subcore_axis_name="s")
  out_type = [jax.ShapeDtypeStruct((NC, NPAD, D), _f32)]
  scratch = [
      pltpu.VMEM((NCHUNK, B), jnp.int32),
      pltpu.VMEM((NCHUNK, B), jnp.int32),
      pltpu.VMEM((B, D), _f32),
      pltpu.VMEM((B, D), _f32),
      pltpu.VMEM_SHARED((NPAD, D), _f32),
      pltpu.VMEM_SHARED((NPAD, D), _f32),
      pltpu.SemaphoreType.DMA,
      pltpu.SemaphoreType.DMA,
  ]
  if with_deg:
    out_type.append(jax.ShapeDtypeStruct((NC, NPAD), _f32))
    scratch += [
        pltpu.VMEM((128,), _f32),
        pltpu.VMEM_SHARED((NPAD,), _f32),
        pltpu.SemaphoreType.DMA,
    ]

  def body(table_hbm, ei_hbm, zeros_hbm, *rest):
    if with_deg:
      (zd_hbm, acc_out, deg_out, src_v, dst_v, rows0, rows1, p_sh, acc_sh,
       sem0, sem1, ones_v, deg_sh, semd) = rest
    else:
      (acc_out, src_v, dst_v, rows0, rows1, p_sh, acc_sh,
       sem0, sem1) = rest
    c = lax.axis_index("c")
    s = lax.axis_index("s")
    wid = c * NS + s
    row0 = pl.multiple_of(s * RPT, 8)

    stage = pltpu.async_copy(table_hbm.at[pl.ds(row0, RPT)],
                             p_sh.at[pl.ds(row0, RPT)], sem0)
    ldsrc = pltpu.async_copy(ei_hbm.at[0, wid], src_v, sem1)
    pltpu.sync_copy(zeros_hbm, acc_sh.at[pl.ds(row0, RPT)])
    pltpu.sync_copy(ei_hbm.at[1, wid], dst_v)
    if with_deg:
      ones16 = jnp.ones((16,), _f32)
      for k in range(128 // 16):
        ones_v[pl.ds(k * 16, 16)] = ones16
      pltpu.sync_copy(zd_hbm, deg_sh.at[pl.ds(row0, RPT)])
    stage.wait()
    ldsrc.wait()
    plsc.subcore_barrier()

    def g_start(j, buf, sem):
      pltpu.async_copy(p_sh.at[src_v.at[j]], buf, sem)

    def g_wait(j, buf, sem):
      pltpu.make_async_copy(p_sh.at[src_v.at[j]], buf, sem).wait()

    def consume(j, buf):
      pltpu.sync_copy(buf, acc_sh.at[dst_v.at[j]], add=True)
      if with_deg:
        pltpu.async_copy(ones_v.at[pl.ds(0, B)], deg_sh.at[dst_v.at[j]],
                        semd, add=True)

    g_start(0, rows0, sem0)
    g_start(1, rows1, sem1)

    def step(i, carry):
      j0 = 2 * i
      g_wait(j0, rows0, sem0)
      consume(j0, rows0)
      g_start(j0 + 2, rows0, sem0)
      g_wait(j0 + 1, rows1, sem1)
      consume(j0 + 1, rows1)
      g_start(j0 + 3, rows1, sem1)
      return carry

    lax.fori_loop(0, NCHUNK // 2 - 1, step, 0)
    g_wait(NCHUNK - 2, rows0, sem0)
    consume(NCHUNK - 2, rows0)
    g_wait(NCHUNK - 1, rows1, sem1)
    consume(NCHUNK - 1, rows1)
    if with_deg:
      def drain(j, carry):
        pltpu.make_async_copy(ones_v.at[pl.ds(0, B)],
                              deg_sh.at[dst_v.at[j]], semd).wait()
        return carry
      lax.fori_loop(0, NCHUNK, drain, 0)
    plsc.subcore_barrier()

    pltpu.sync_copy(acc_sh.at[pl.ds(row0, RPT)],
                    acc_out.at[c, pl.ds(row0, RPT)])
    if with_deg:
      pltpu.sync_copy(deg_sh.at[pl.ds(row0, RPT)],
                      deg_out.at[c, pl.ds(row0, RPT)])

  return pl.kernel(body, out_type=out_type, mesh=mesh, scratch_types=scratch,
                   compiler_params=pltpu.CompilerParams(
                       use_tc_tiling_on_sc=False))


_seg_sum_l1 = _make_seg_sum(32, with_deg=True)
_seg_sum_l2 = _make_seg_sum(16, with_deg=False)

BN = 2560
_ONES2 = (((0,), (0,)), ((), ()))


def _rdeg_col(deg2):
  ones2 = jnp.ones((2, 1), _f32)
  deg = lax.dot_general(deg2, ones2, _ONES2, preferred_element_type=_f32)
  return 1.0 / jnp.maximum(deg, 1.0)


def _stage1_body(x_ref, wl_ref, wr_ref, t_o, r_o):
  xv = x_ref[...]
  t_o[...] = jnp.dot(xv, wl_ref[...], preferred_element_type=_f32)
  r_o[...] = jnp.dot(xv, wr_ref[...], preferred_element_type=_f32)


def _mid_body(acc_ref, deg_ref, p1r_ref, b1_ref, w2l_ref, w2r_ref,
              t_o, p2r_o):
  acc = acc_ref[...]
  s1 = acc[0] + acc[1]
  rdeg = _rdeg_col(deg_ref[...])
  h = s1 * rdeg + b1_ref[...] + p1r_ref[...]
  h = jnp.maximum(h, 0.0)
  t_o[...] = jnp.dot(h, w2l_ref[...], preferred_element_type=_f32)
  p2r_o[...] = jnp.dot(h, w2r_ref[...], preferred_element_type=_f32)


def _dec_body(acc_ref, deg_ref, p2r_ref, b2_ref, wd_ref, bd_ref,
              xr_o, z_o):
  acc = acc_ref[...]
  rdeg = _rdeg_col(deg_ref[...])
  z = (acc[0] + acc[1]) * rdeg + b2_ref[...] + p2r_ref[...]
  z_o[...] = z
  xr_o[...] = jnp.dot(z, wd_ref[...], preferred_element_type=_f32) + bd_ref[...]


def _row_blocks(width, bn=BN):
  return pl.BlockSpec((bn, width), lambda i: (i, 0))


def _acc_blocks(width, bn=BN):
  return pl.BlockSpec((NC, bn, width), lambda i: (0, i, 0))


def _deg_blocks(bn=BN):
  return pl.BlockSpec((NC, bn), lambda i: (0, i))


def _full(shape):
  return pl.BlockSpec(shape, lambda i: tuple(0 for _ in shape))


def kernel(x, edge_index, W1_l, b1_l, W1_r, W2_l, b2_l, W2_r, W_dec, b_dec):
  ei = lax.convert_element_type(edge_index, jnp.int32).reshape(
      2, NW, NCHUNK, B)
  z1 = jnp.zeros((RPT, 32), _f32)
  z2 = jnp.zeros((RPT, 16), _f32)

  table1, p1r = pl.pallas_call(
      _stage1_body,
      grid=(NPAD // BN,),
      in_specs=[_row_blocks(128), _full((128, 32)), _full((128, 32))],
      out_specs=[_row_blocks(32), _row_blocks(32)],
      out_shape=[jax.ShapeDtypeStruct((NPAD, 32), _f32),
                 jax.ShapeDtypeStruct((NPAD, 32), _f32)],
  )(x, W1_l.T, W1_r.T)

  acc1, degp = _seg_sum_l1(table1, ei, z1, jnp.zeros((RPT,), _f32))

  table2, p2r = pl.pallas_call(
      _mid_body,
      grid=(NPAD // BN,),
      in_specs=[_acc_blocks(32), _deg_blocks(), _row_blocks(32),
                _full((1, 32)), _full((32, 16)), _full((32, 16))],
      out_specs=[_row_blocks(16), _row_blocks(16)],
      out_shape=[jax.ShapeDtypeStruct((NPAD, 16), _f32),
                 jax.ShapeDtypeStruct((NPAD, 16), _f32)],
  )(acc1, degp, p1r, b1_l[None, :], W2_l.T, W2_r.T)

  (acc2,) = _seg_sum_l2(table2, ei, z2)

  x_recon, z = pl.pallas_call(
      _dec_body,
      grid=(NPAD // BN,),
      in_specs=[_acc_blocks(16), _deg_blocks(), _row_blocks(16),
                _full((1, 16)), _full((16, 128)), _full((1, 128))],
      out_specs=[_row_blocks(128), _row_blocks(16)],
      out_shape=[jax.ShapeDtypeStruct((N, 128), _f32),
                 jax.ShapeDtypeStruct((N, 16), _f32)],
  )(acc2, degp, p2r, b2_l[None, :], W_dec.T, b_dec[None, :])

  return (x_recon, z)


# speedup vs baseline: 1.2415x; 1.0254x over previous
import jax
import jax.numpy as jnp
from jax import lax
from jax.experimental import pallas as pl
from jax.experimental.pallas import tpu as pltpu
from jax.experimental.pallas import tpu_sc as plsc

N = 10000
NPAD = 10240
E = 320000
NC = 2
NS = 16
NW = NC * NS
RPT = NPAD // NS
EPW = E // NW
B = 125
NCHUNK = EPW // B

_f32 = jnp.float32


def _make_seg_sum(D, with_deg):
  mesh = plsc.VectorSubcoreMesh(core_axis_name="c", subcore_axis_name="s")
  out_type = [jax.ShapeDtypeStruct((NC, NPAD, D), _f32)]
  scratch = [
      pltpu.VMEM((NCHUNK, B), jnp.int32),
      pltpu.VMEM((NCHUNK, B), jnp.int32),
      pltpu.VMEM((B, D), _f32),
      pltpu.VMEM((B, D), _f32),
      pltpu.VMEM_SHARED((NPAD, D), _f32),
      pltpu.VMEM_SHARED((NPAD, D), _f32),
      pltpu.SemaphoreType.DMA,
      pltpu.SemaphoreType.DMA,
  ]
  if with_deg:
    out_type.append(jax.ShapeDtypeStruct((NC, NPAD), _f32))
    scratch += [
        pltpu.VMEM((128,), _f32),
        pltpu.VMEM_SHARED((NPAD,), _f32),
        pltpu.SemaphoreType.DMA,
    ]

  def body(table_hbm, ei_hbm, zeros_hbm, *rest):
    if with_deg:
      (zd_hbm, acc_out, deg_out, src_v, dst_v, rows0, rows1, p_sh, acc_sh,
       sem0, sem1, ones_v, deg_sh, semd) = rest
    else:
      (acc_out, src_v, dst_v, rows0, rows1, p_sh, acc_sh,
       sem0, sem1) = rest
    c = lax.axis_index("c")
    s = lax.axis_index("s")
    wid = c * NS + s
    row0 = pl.multiple_of(s * RPT, 8)

    stage = pltpu.async_copy(table_hbm.at[pl.ds(row0, RPT)],
                             p_sh.at[pl.ds(row0, RPT)], sem0)
    ldsrc = pltpu.async_copy(ei_hbm.at[0, wid], src_v, sem1)
    pltpu.sync_copy(zeros_hbm, acc_sh.at[pl.ds(row0, RPT)])
    pltpu.sync_copy(ei_hbm.at[1, wid], dst_v)
    if with_deg:
      ones16 = jnp.ones((16,), _f32)
      for k in range(128 // 16):
        ones_v[pl.ds(k * 16, 16)] = ones16
      pltpu.sync_copy(zd_hbm, deg_sh.at[pl.ds(row0, RPT)])
    stage.wait()
    ldsrc.wait()
    plsc.subcore_barrier()

    def g_start(j, buf, sem):
      pltpu.async_copy(p_sh.at[src_v.at[j]], buf, sem)

    def g_wait(j, buf, sem):
      pltpu.make_async_copy(p_sh.at[src_v.at[j]], buf, sem).wait()

    def consume(j, buf):
      pltpu.sync_copy(buf, acc_sh.at[dst_v.at[j]], add=True)
      if with_deg:
        pltpu.async_copy(ones_v.at[pl.ds(0, B)], deg_sh.at[dst_v.at[j]],
                        semd, add=True)

    g_start(0, rows0, sem0)
    g_start(1, rows1, sem1)

    def step(i, carry):
      j0 = 2 * i
      g_wait(j0, rows0, sem0)
      consume(j0, rows0)
      g_start(j0 + 2, rows0, sem0)
      g_wait(j0 + 1, rows1, sem1)
      consume(j0 + 1, rows1)
      g_start(j0 + 3, rows1, sem1)
      return carry

    lax.fori_loop(0, NCHUNK // 2 - 1, step, 0)
    g_wait(NCHUNK - 2, rows0, sem0)
    consume(NCHUNK - 2, rows0)
    g_wait(NCHUNK - 1, rows1, sem1)
    consume(NCHUNK - 1, rows1)
    if with_deg:
      def drain(j, carry):
        pltpu.make_async_copy(ones_v.at[pl.ds(0, B)],
                              deg_sh.at[dst_v.at[j]], semd).wait()
        return carry
      lax.fori_loop(0, NCHUNK, drain, 0)
    plsc.subcore_barrier()

    pltpu.sync_copy(acc_sh.at[pl.ds(row0, RPT)],
                    acc_out.at[c, pl.ds(row0, RPT)])
    if with_deg:
      pltpu.sync_copy(deg_sh.at[pl.ds(row0, RPT)],
                      deg_out.at[c, pl.ds(row0, RPT)])

  return pl.kernel(body, out_type=out_type, mesh=mesh, scratch_types=scratch,
                   compiler_params=pltpu.CompilerParams(
                       use_tc_tiling_on_sc=False))


_seg_sum_l1 = _make_seg_sum(32, with_deg=True)
_seg_sum_l2 = _make_seg_sum(16, with_deg=False)

BN = 5120
_ONES2 = (((0,), (0,)), ((), ()))


def _rdeg_col(deg2):
  ones2 = jnp.ones((2, 1), _f32)
  deg = lax.dot_general(deg2, ones2, _ONES2, preferred_element_type=_f32)
  return 1.0 / jnp.maximum(deg, 1.0)


def _stage1_body(x_ref, wl_ref, wr_ref, t_o, r_o):
  xv = x_ref[...]
  t_o[...] = jnp.dot(xv, wl_ref[...], preferred_element_type=_f32)
  r_o[...] = jnp.dot(xv, wr_ref[...], preferred_element_type=_f32)


def _mid_body(acc_ref, deg_ref, p1r_ref, b1_ref, w2l_ref, w2r_ref,
              t_o, p2r_o):
  acc = acc_ref[...]
  s1 = acc[0] + acc[1]
  rdeg = _rdeg_col(deg_ref[...])
  h = s1 * rdeg + b1_ref[...] + p1r_ref[...]
  h = jnp.maximum(h, 0.0)
  t_o[...] = jnp.dot(h, w2l_ref[...], preferred_element_type=_f32)
  p2r_o[...] = jnp.dot(h, w2r_ref[...], preferred_element_type=_f32)


def _dec_body(acc_ref, deg_ref, p2r_ref, b2_ref, wd_ref, bd_ref,
              xr_o, z_o):
  acc = acc_ref[...]
  rdeg = _rdeg_col(deg_ref[...])
  z = (acc[0] + acc[1]) * rdeg + b2_ref[...] + p2r_ref[...]
  z_o[...] = z
  xr_o[...] = jnp.dot(z, wd_ref[...], preferred_element_type=_f32) + bd_ref[...]


def _row_blocks(width, bn=BN):
  return pl.BlockSpec((bn, width), lambda i: (i, 0))


def _acc_blocks(width, bn=BN):
  return pl.BlockSpec((NC, bn, width), lambda i: (0, i, 0))


def _deg_blocks(bn=BN):
  return pl.BlockSpec((NC, bn), lambda i: (0, i))


def _full(shape):
  return pl.BlockSpec(shape, lambda i: tuple(0 for _ in shape))


def kernel(x, edge_index, W1_l, b1_l, W1_r, W2_l, b2_l, W2_r, W_dec, b_dec):
  ei = lax.convert_element_type(edge_index, jnp.int32).reshape(
      2, NW, NCHUNK, B)
  z1 = jnp.zeros((RPT, 32), _f32)
  z2 = jnp.zeros((RPT, 16), _f32)

  table1, p1r = pl.pallas_call(
      _stage1_body,
      grid=(NPAD // BN,),
      in_specs=[_row_blocks(128), _full((128, 32)), _full((128, 32))],
      out_specs=[_row_blocks(32), _row_blocks(32)],
      out_shape=[jax.ShapeDtypeStruct((NPAD, 32), _f32),
                 jax.ShapeDtypeStruct((NPAD, 32), _f32)],
  )(x, W1_l.T, W1_r.T)

  acc1, degp = _seg_sum_l1(table1, ei, z1, jnp.zeros((RPT,), _f32))

  table2, p2r = pl.pallas_call(
      _mid_body,
      grid=(NPAD // BN,),
      in_specs=[_acc_blocks(32), _deg_blocks(), _row_blocks(32),
                _full((1, 32)), _full((32, 16)), _full((32, 16))],
      out_specs=[_row_blocks(16), _row_blocks(16)],
      out_shape=[jax.ShapeDtypeStruct((NPAD, 16), _f32),
                 jax.ShapeDtypeStruct((NPAD, 16), _f32)],
  )(acc1, degp, p1r, b1_l[None, :], W2_l.T, W2_r.T)

  (acc2,) = _seg_sum_l2(table2, ei, z2)

  x_recon, z = pl.pallas_call(
      _dec_body,
      grid=(NPAD // BN,),
      in_specs=[_acc_blocks(16), _deg_blocks(), _row_blocks(16),
                _full((1, 16)), _full((16, 128)), _full((1, 128))],
      out_specs=[_row_blocks(128), _row_blocks(16)],
      out_shape=[jax.ShapeDtypeStruct((N, 128), _f32),
                 jax.ShapeDtypeStruct((N, 16), _f32)],
  )(acc2, degp, p2r, b2_l[None, :], W_dec.T, b_dec[None, :])

  return (x_recon, z)
